# baseline (speedup 1.0000x reference)
import functools

import jax
import jax.numpy as jnp
from jax import lax
from jax.experimental import pallas as pl
from jax.experimental.pallas import tpu as pltpu
from jax.experimental.pallas import tpu_sc as plsc

_NC, _NS, _L = 2, 16, 16
_NW = _NC * _NS

_C = 4096
_G = _C // _L


def _tec_body(nchunk, dims, x_hbm, tab_hbm, par_hbm, out_hbm,
              xv, pv, idx0, idx1, vals0, vals1, w0, w1, outv,
              sem0, sem1):
    gx, gy, gz = dims
    sx, sy = gy * gz, gz
    poffs = (0, sy, sx, sx + sy)
    wid = lax.axis_index("s") * _NC + lax.axis_index("c")
    ppw = nchunk * _C
    n_total = ppw * _NW

    pltpu.sync_copy(par_hbm, pv)

    def front(ci, idxv, wv, sem):
        base = wid * ppw + ci * _C
        pltpu.sync_copy(x_hbm.at[pl.ds(base * 3, _C * 3)], xv)

        lxv = pv[pl.ds(0, _L)]
        lyv = pv[pl.ds(_L, _L)]
        lzv = pv[pl.ds(2 * _L, _L)]
        rinv = 1.0 / pv[pl.ds(3 * _L, _L)]

        lanes3 = lax.iota(jnp.int32, _L) * 3

        @pl.loop(0, _G)
        def compute(j):
            p0 = j * _L
            rows = lanes3 + p0 * 3
            px = plsc.load_gather(xv, [rows])
            py = plsc.load_gather(xv, [rows + 1])
            pz = plsc.load_gather(xv, [rows + 2])
            ix = (px - lxv) * rinv
            iy = (py - lyv) * rinv
            iz = (pz - lzv) * rinv
            valid = ((jnp.minimum(jnp.minimum(ix, iy), iz) >= 0.0)
                     & (ix <= gx - 1.0) & (iy <= gy - 1.0)
                     & (iz <= gz - 1.0))
            x0 = jnp.clip(ix.astype(jnp.int32), 0, gx - 2)
            y0 = jnp.clip(iy.astype(jnp.int32), 0, gy - 2)
            z0 = jnp.clip(iz.astype(jnp.int32), 0, gz - 2)
            fx = ix - x0.astype(jnp.float32)
            fy = iy - y0.astype(jnp.float32)
            fz = iz - z0.astype(jnp.float32)
            vf = jnp.where(valid, jnp.float32(1.0), jnp.float32(0.0))
            flat = x0 * sx + y0 * sy + z0
            for k in range(4):
                idxv[pl.ds(k * _C + p0, _L)] = flat + poffs[k]
            wv[pl.ds(0 * _C + p0, _L)] = fx
            wv[pl.ds(1 * _C + p0, _L)] = fy
            wv[pl.ds(2 * _C + p0, _L)] = fz
            wv[pl.ds(3 * _C + p0, _L)] = vf

        valsv = vals0 if sem is sem0 else vals1
        pltpu.async_copy(tab_hbm.at[idxv], valsv, sem)

    def back(ci, idxv, valsv, wv, sem):
        base = wid * ppw + ci * _C

        pltpu.make_async_copy(tab_hbm.at[idxv], valsv, sem).wait()

        @pl.loop(0, _G)
        def interp(j):
            p0 = j * _L
            cz = []
            for k in range(4):
                pair = valsv[pl.ds(k * _C + p0, _L)]
                z0v = plsc.bitcast(pair << 16, jnp.float32)
                z1v = plsc.bitcast(pair & jnp.int32(-65536), jnp.float32)
                cz.append((z0v, z1v))
            fx = wv[pl.ds(0 * _C + p0, _L)]
            fy = wv[pl.ds(1 * _C + p0, _L)]
            fz = wv[pl.ds(2 * _C + p0, _L)]
            vf = wv[pl.ds(3 * _C + p0, _L)]
            c00 = cz[0][0] * (1 - fz) + cz[0][1] * fz
            c01 = cz[1][0] * (1 - fz) + cz[1][1] * fz
            c10 = cz[2][0] * (1 - fz) + cz[2][1] * fz
            c11 = cz[3][0] * (1 - fz) + cz[3][1] * fz
            c0 = c00 * (1 - fy) + c01 * fy
            c1 = c10 * (1 - fy) + c11 * fy
            outv[pl.ds(p0, _L)] = (c0 * (1 - fx) + c1 * fx) * vf

        pltpu.sync_copy(outv, out_hbm.at[pl.ds(base, _C)])

    front(0, idx0, w0, sem0)

    @pl.loop(0, nchunk, step=2)
    def chunk_loop(ci):
        front(ci + 1, idx1, w1, sem1)
        back(ci, idx0, vals0, w0, sem0)

        @pl.when(ci + 2 < nchunk)
        def _():
            front(ci + 2, idx0, w0, sem0)

        back(ci + 1, idx1, vals1, w1, sem1)


def _pack_pairs_body(g_ref, t_ref):
    gi = jax.lax.bitcast_convert_type(g_ref[...], jnp.int32)
    pay = jax.lax.shift_right_logical(
        gi + jnp.int32(0x7FFF) + (jax.lax.shift_right_logical(gi, 16) & 1),
        16)
    pay_next = pltpu.roll(pay, gi.shape[1] - 1, axis=1)
    t_ref[...] = (pay | (pay_next << 16)).reshape(-1)


def kernel(x, grid, lower, resolution):
    n = x.shape[0]
    dims = grid.shape
    assert n % (_NW * _C) == 0
    nchunk = n // (_NW * _C)
    assert nchunk % 2 == 0

    rows, gz = dims[0] * dims[1], dims[2]
    br = 2048
    while rows % br:
        br //= 2
    g2 = grid.reshape(rows, gz)
    table = pl.pallas_call(
        _pack_pairs_body,
        grid=(rows // br,),
        in_specs=[pl.BlockSpec((br, gz), lambda i: (i, 0))],
        out_specs=pl.BlockSpec((br * gz,), lambda i: (i,)),
        out_shape=jax.ShapeDtypeStruct((rows * gz,), jnp.int32),
    )(g2)

    params = jnp.concatenate([
        jnp.full((_L,), lower[0], jnp.float32),
        jnp.full((_L,), lower[1], jnp.float32),
        jnp.full((_L,), lower[2], jnp.float32),
        jnp.full((_L,), resolution, jnp.float32),
    ])

    mesh = plsc.VectorSubcoreMesh(core_axis_name="c", subcore_axis_name="s",
                                  num_cores=_NC, num_subcores=_NS)
    f = pl.kernel(
        functools.partial(_tec_body, nchunk, dims),
        out_type=jax.ShapeDtypeStruct((n,), jnp.float32),
        mesh=mesh,
        compiler_params=pltpu.CompilerParams(needs_layout_passes=False),
        scratch_types=[
            pltpu.VMEM((3 * _C,), jnp.float32),
            pltpu.VMEM((4 * _L,), jnp.float32),
            pltpu.VMEM((4 * _C,), jnp.int32),
            pltpu.VMEM((4 * _C,), jnp.int32),
            pltpu.VMEM((4 * _C,), jnp.int32),
            pltpu.VMEM((4 * _C,), jnp.int32),
            pltpu.VMEM((4 * _C,), jnp.float32),
            pltpu.VMEM((4 * _C,), jnp.float32),
            pltpu.VMEM((_C,), jnp.float32),
            pltpu.SemaphoreType.DMA,
            pltpu.SemaphoreType.DMA,
        ],
    )
    return f(x.reshape(-1), table, params)


# speedup vs baseline: 5.5623x; 5.5623x over previous
import functools

import jax
import jax.numpy as jnp
from jax import lax
from jax.experimental import pallas as pl
from jax.experimental.pallas import tpu as pltpu
from jax.experimental.pallas import tpu_sc as plsc

_NC, _NS, _L = 2, 16, 16
_NW = _NC * _NS

_C = 2048
_G = _C // _L


def _tec_body(nchunk, dims, x_hbm, tab_hbm, par_hbm, out_hbm,
              x0v, x1v, x2v, pv, idx0, idx1, vals0, vals1, w0, w1, outv,
              sem0, sem1):
    gx, gy, gz = dims
    sx, sy = gy * gz, gz
    poffs = (0, sy, sx, sx + sy)
    wid = lax.axis_index("s") * _NC + lax.axis_index("c")
    ppw = nchunk * _C
    n_total = ppw * _NW

    pltpu.sync_copy(par_hbm, pv)

    def front(ci, idxv, wv, sem):
        base = wid * ppw + ci * _C
        pltpu.sync_copy(x_hbm.at[pl.ds(base, _C)], x0v)
        pltpu.sync_copy(x_hbm.at[pl.ds(base + n_total, _C)], x1v)
        pltpu.sync_copy(x_hbm.at[pl.ds(base + 2 * n_total, _C)], x2v)

        lxv = pv[pl.ds(0, _L)]
        lyv = pv[pl.ds(_L, _L)]
        lzv = pv[pl.ds(2 * _L, _L)]
        resv = pv[pl.ds(3 * _L, _L)]

        @pl.loop(0, _G)
        def compute(j):
            p0 = j * _L
            px = x0v[pl.ds(p0, _L)]
            py = x1v[pl.ds(p0, _L)]
            pz = x2v[pl.ds(p0, _L)]
            ix = (px - lxv) / resv
            iy = (py - lyv) / resv
            iz = (pz - lzv) / resv
            valid = ((ix >= 0.0) & (ix <= gx - 1.0)
                     & (iy >= 0.0) & (iy <= gy - 1.0)
                     & (iz >= 0.0) & (iz <= gz - 1.0))
            x0 = jnp.clip(ix.astype(jnp.int32), 0, gx - 2)
            y0 = jnp.clip(iy.astype(jnp.int32), 0, gy - 2)
            z0 = jnp.clip(iz.astype(jnp.int32), 0, gz - 2)
            fx = ix - x0.astype(jnp.float32)
            fy = iy - y0.astype(jnp.float32)
            fz = iz - z0.astype(jnp.float32)
            vf = jnp.where(valid, jnp.float32(1.0), jnp.float32(0.0))
            flat = x0 * sx + y0 * sy + z0
            for k in range(4):
                idxv[pl.ds(k * _C + p0, _L)] = flat + poffs[k]
            wv[pl.ds(0 * _C + p0, _L)] = fx
            wv[pl.ds(1 * _C + p0, _L)] = fy
            wv[pl.ds(2 * _C + p0, _L)] = fz
            wv[pl.ds(3 * _C + p0, _L)] = vf

        valsv = vals0 if sem is sem0 else vals1
        pltpu.async_copy(tab_hbm.at[idxv], valsv, sem)

    def back(ci, idxv, valsv, wv, sem):
        base = wid * ppw + ci * _C

        pltpu.make_async_copy(tab_hbm.at[idxv], valsv, sem).wait()

        @pl.loop(0, _G)
        def interp(j):
            p0 = j * _L
            cz = []
            for k in range(4):
                pair = valsv[pl.ds(k * _C + p0, _L)]
                z0v = plsc.bitcast(pair << 16, jnp.float32)
                z1v = plsc.bitcast(pair & jnp.int32(-65536), jnp.float32)
                cz.append((z0v, z1v))
            fx = wv[pl.ds(0 * _C + p0, _L)]
            fy = wv[pl.ds(1 * _C + p0, _L)]
            fz = wv[pl.ds(2 * _C + p0, _L)]
            vf = wv[pl.ds(3 * _C + p0, _L)]
            c00 = cz[0][0] * (1 - fz) + cz[0][1] * fz
            c01 = cz[1][0] * (1 - fz) + cz[1][1] * fz
            c10 = cz[2][0] * (1 - fz) + cz[2][1] * fz
            c11 = cz[3][0] * (1 - fz) + cz[3][1] * fz
            c0 = c00 * (1 - fy) + c01 * fy
            c1 = c10 * (1 - fy) + c11 * fy
            outv[pl.ds(p0, _L)] = (c0 * (1 - fx) + c1 * fx) * vf

        pltpu.sync_copy(outv, out_hbm.at[pl.ds(base, _C)])

    front(0, idx0, w0, sem0)

    @pl.loop(0, nchunk, step=2)
    def chunk_loop(ci):
        front(ci + 1, idx1, w1, sem1)
        back(ci, idx0, vals0, w0, sem0)

        @pl.when(ci + 2 < nchunk)
        def _():
            front(ci + 2, idx0, w0, sem0)

        back(ci + 1, idx1, vals1, w1, sem1)


def _pack_pairs_body(g_ref, t_ref):
    gi = jax.lax.bitcast_convert_type(g_ref[...], jnp.int32)
    pay = jax.lax.shift_right_logical(
        gi + jnp.int32(0x7FFF) + (jax.lax.shift_right_logical(gi, 16) & 1),
        16)
    pay_next = pltpu.roll(pay, gi.shape[1] - 1, axis=1)
    t_ref[...] = (pay | (pay_next << 16)).reshape(-1)


def kernel(x, grid, lower, resolution):
    n = x.shape[0]
    dims = grid.shape
    assert n % (_NW * _C) == 0
    nchunk = n // (_NW * _C)
    assert nchunk % 2 == 0

    rows, gz = dims[0] * dims[1], dims[2]
    br = 2048
    while rows % br:
        br //= 2
    g2 = grid.reshape(rows, gz)
    table = pl.pallas_call(
        _pack_pairs_body,
        grid=(rows // br,),
        in_specs=[pl.BlockSpec((br, gz), lambda i: (i, 0))],
        out_specs=pl.BlockSpec((br * gz,), lambda i: (i,)),
        out_shape=jax.ShapeDtypeStruct((rows * gz,), jnp.int32),
    )(g2)

    params = jnp.concatenate([
        jnp.full((_L,), lower[0], jnp.float32),
        jnp.full((_L,), lower[1], jnp.float32),
        jnp.full((_L,), lower[2], jnp.float32),
        jnp.full((_L,), resolution, jnp.float32),
    ])

    mesh = plsc.VectorSubcoreMesh(core_axis_name="c", subcore_axis_name="s",
                                  num_cores=_NC, num_subcores=_NS)
    f = pl.kernel(
        functools.partial(_tec_body, nchunk, dims),
        out_type=jax.ShapeDtypeStruct((n,), jnp.float32),
        mesh=mesh,
        compiler_params=pltpu.CompilerParams(needs_layout_passes=False),
        scratch_types=[
            pltpu.VMEM((_C,), jnp.float32),
            pltpu.VMEM((_C,), jnp.float32),
            pltpu.VMEM((_C,), jnp.float32),
            pltpu.VMEM((4 * _L,), jnp.float32),
            pltpu.VMEM((4 * _C,), jnp.int32),
            pltpu.VMEM((4 * _C,), jnp.int32),
            pltpu.VMEM((4 * _C,), jnp.int32),
            pltpu.VMEM((4 * _C,), jnp.int32),
            pltpu.VMEM((4 * _C,), jnp.float32),
            pltpu.VMEM((4 * _C,), jnp.float32),
            pltpu.VMEM((_C,), jnp.float32),
            pltpu.SemaphoreType.DMA,
            pltpu.SemaphoreType.DMA,
        ],
    )
    return f(x.T.reshape(-1), table, params)
